# Initial kernel scaffold; baseline (speedup 1.0000x reference)
import jax
import jax.numpy as jnp
from jax.experimental import pallas as pl


def kernel(atomic_charges, distances, idx_i, idx_j):
    raise NotImplementedError("write your pallas kernel here")


# speedup vs baseline: 258.0131x; 258.0131x over previous
import functools

import jax
import jax.numpy as jnp
from jax import lax
from jax.experimental import pallas as pl
from jax.experimental.pallas import tpu as pltpu
from jax.experimental.pallas import tpu_sc as plsc

NC = 2
NS = 16
NW = NC * NS
LANES = 16
CHUNK = 2048
ROWS = CHUNK // 128

KEHALF = 7.199822675975274
CUTOFF = 12.0
CUTOFF_SQ = CUTOFF * CUTOFF
CUTOFF_SR = 8.0


def _edge_energy(d, qi, qj):
    c_inv_cut2 = 1.0 / CUTOFF_SQ
    u = d * d
    a = u + 1.0
    bits = lax.bitcast_convert_type(a, jnp.int32)
    y = lax.bitcast_convert_type(jnp.int32(0x5F3759DF) - (bits >> 1), jnp.float32)
    h = 0.5 * a
    y = y * (1.5 - h * y * y)
    y = y * (1.5 - h * y * y)
    y = y * (1.5 - h * y * y)
    e_sh = y * (1.0 + a * c_inv_cut2) - (2.0 / CUTOFF)
    e_or = 1.0 / d + d * c_inv_cut2 - (2.0 / CUTOFF)
    x = jnp.minimum(d, CUTOFF_SR) * (1.0 / CUTOFF_SR)
    x2 = x * x
    poly = 6.0 * x2 - 15.0 * x + 10.0
    sw_off = 1.0 - (x2 * x) * poly
    t = e_or + sw_off * (e_sh - e_or)
    e = (KEHALF * qi) * qj * t
    return jnp.where(d <= CUTOFF, e, jnp.zeros_like(e))


def _make_sc_kernel(n_nodes, n_edges):
    assert n_edges % CHUNK == 0
    acc_pad = ((n_nodes + NS * 128 - 1) // (NS * 128)) * (NS * 128)
    slc = acc_pad // NS
    nchunks = n_edges // CHUNK
    base_ch = nchunks // NW
    rem_ch = nchunks % NW

    mesh = plsc.VectorSubcoreMesh(
        core_axis_name="c", subcore_axis_name="s", num_cores=NC, num_subcores=NS
    )

    def body(q_hbm, d_hbm, ii_hbm, ij_hbm, out_hbm,
             tab, dbuf, iibuf, ijbuf, ebuf, zbuf, acc):
        cid = lax.axis_index("c")
        sid = lax.axis_index("s")
        wid = sid * NC + cid

        pltpu.sync_copy(q_hbm, tab)

        zz = jnp.zeros((LANES,), jnp.float32)

        def _z(i, carry):
            zbuf[pl.ds(i * LANES, LANES)] = zz
            return carry

        lax.fori_loop(0, slc // LANES, _z, 0)
        pltpu.sync_copy(zbuf, acc.at[pl.ds(sid * slc, slc)])
        plsc.subcore_barrier()

        nch = base_ch + jnp.where(wid < rem_ch, 1, 0)

        def _chunk(g, carry):
            c = wid + NW * g
            base = c * CHUNK
            rbase = c * ROWS
            pltpu.sync_copy(d_hbm.at[pl.ds(base, CHUNK)], dbuf)
            pltpu.sync_copy(ii_hbm.at[pl.ds(rbase, ROWS)], iibuf)
            pltpu.sync_copy(ij_hbm.at[pl.ds(base, CHUNK)], ijbuf)

            def _vec(r, carry2):
                for t in range(128 // LANES):
                    col = t * LANES
                    off = r * 128 + col
                    d = dbuf[pl.ds(off, LANES)]
                    ii = iibuf[r, pl.ds(col, LANES)]
                    ij = ijbuf[pl.ds(off, LANES)]
                    qi = plsc.load_gather(tab, [ii])
                    qj = plsc.load_gather(tab, [ij])
                    ebuf[r, pl.ds(col, LANES)] = _edge_energy(d, qi, qj)
                return carry2

            lax.fori_loop(0, ROWS, _vec, 0)

            def _scat(b, carry2):
                pltpu.sync_copy(ebuf.at[b], acc.at[iibuf.at[b]], add=True)
                return carry2

            lax.fori_loop(0, ROWS, _scat, 0)
            return carry

        lax.fori_loop(0, nch, _chunk, 0)

        plsc.subcore_barrier()
        pltpu.sync_copy(acc.at[pl.ds(sid * slc, slc)],
                        out_hbm.at[cid * NS + sid, 0])

    return pl.kernel(
        body,
        out_type=jax.ShapeDtypeStruct((NW, 1, slc), jnp.float32),
        mesh=mesh,
        compiler_params=pltpu.CompilerParams(needs_layout_passes=False),
        scratch_types=[
            pltpu.VMEM((n_nodes,), jnp.float32),
            pltpu.VMEM((CHUNK,), jnp.float32),
            pltpu.VMEM((ROWS, 128), jnp.int32),
            pltpu.VMEM((CHUNK,), jnp.int32),
            pltpu.VMEM((ROWS, 128), jnp.float32),
            pltpu.VMEM((slc,), jnp.float32),
            pltpu.VMEM_SHARED((acc_pad,), jnp.float32),
        ],
    )


@jax.jit
def kernel(atomic_charges, distances, idx_i, idx_j):
    n_nodes = atomic_charges.shape[0]
    n_edges = distances.shape[0]
    q = atomic_charges.astype(jnp.float32)
    d = distances.astype(jnp.float32)
    ii = idx_i.astype(jnp.int32).reshape(n_edges // 128, 128)
    ij = idx_j.astype(jnp.int32)
    partials = _make_sc_kernel(n_nodes, n_edges)(q, d, ii, ij)
    per_core = partials.reshape(NC, NS * (partials.shape[2]))
    return (per_core[0] + per_core[1])[:n_nodes]


# speedup vs baseline: 713.6677x; 2.7660x over previous
import functools

import jax
import jax.numpy as jnp
from jax import lax
from jax.experimental import pallas as pl
from jax.experimental.pallas import tpu as pltpu
from jax.experimental.pallas import tpu_sc as plsc

NC = 2
NS = 16
NW = NC * NS
LANES = 16
CHUNK = 1024
ROWS = CHUNK // 128
NBUF = 3

KEHALF = 7.199822675975274
CUTOFF = 12.0
CUTOFF_SQ = CUTOFF * CUTOFF
CUTOFF_SR = 8.0


def _edge_energy(d, qi, qj):
    A = KEHALF
    C = KEHALF / CUTOFF_SQ
    S = 2.0 * KEHALF / CUTOFF
    a = d * d + 1.0
    bits = lax.bitcast_convert_type(a, jnp.int32)
    y = lax.bitcast_convert_type(jnp.int32(0x5F3759DF) - (bits >> 1), jnp.float32)
    h = 0.5 * a
    y = y * (1.5 - h * y * y)
    y = y * (1.5 - h * y * y)
    f_sh = y * (A + a * C)
    f_or = A / d + d * C
    x = jnp.minimum(d, CUTOFF_SR) * (1.0 / CUTOFF_SR)
    x2 = x * x
    poly = 6.0 * x2 - 15.0 * x + 10.0
    sw_off = 1.0 - (x2 * x) * poly
    t = (f_or - S) + sw_off * (f_sh - f_or)
    e = qi * qj * t
    return jnp.where(d <= CUTOFF, e, jnp.zeros_like(e))


def _make_sc_kernel(n_nodes, n_edges):
    assert n_edges % CHUNK == 0
    acc_pad = ((n_nodes + NS * 128 - 1) // (NS * 128)) * (NS * 128)
    slc = acc_pad // NS
    zlen = slc // 4
    assert zlen % LANES == 0
    nchunks = n_edges // CHUNK
    base_ch = nchunks // NW
    rem_ch = nchunks % NW

    mesh = plsc.VectorSubcoreMesh(
        core_axis_name="c", subcore_axis_name="s", num_cores=NC, num_subcores=NS
    )

    def body(q_hbm, d_hbm, ii_hbm, ij_hbm, out_hbm,
             tab, dbufs, iibufs, ijbufs, ebufs, zbuf, acc, in_sems, sc_sems):
        cid = lax.axis_index("c")
        sid = lax.axis_index("s")
        wid = sid * NC + cid

        pltpu.sync_copy(q_hbm, tab)

        zz = jnp.zeros((LANES,), jnp.float32)

        def _z(i, carry):
            zbuf[pl.ds(i * LANES, LANES)] = zz
            return carry

        lax.fori_loop(0, zlen // LANES, _z, 0)
        for p in range(4):
            pltpu.sync_copy(zbuf, acc.at[pl.ds(sid * slc + p * zlen, zlen)])
        plsc.subcore_barrier()

        nch = base_ch + jnp.where(wid < rem_ch, 1, 0)

        def _in_copies(c, st):
            base = c * CHUNK
            rbase = c * ROWS
            return (
                pltpu.make_async_copy(
                    d_hbm.at[pl.ds(base, CHUNK)], dbufs[st], in_sems[st]),
                pltpu.make_async_copy(
                    ii_hbm.at[pl.ds(rbase, ROWS)], iibufs[st], in_sems[st]),
                pltpu.make_async_copy(
                    ij_hbm.at[pl.ds(base, CHUNK)], ijbufs[st], in_sems[st]),
            )

        def _fire_inputs(c, st):
            for cp in _in_copies(c, st):
                cp.start()

        def _wait_inputs(c, st):
            for cp in _in_copies(c, st):
                cp.wait()

        def _drain_scatter(st):
            for b in range(ROWS):
                pltpu.make_async_copy(
                    ebufs[st].at[b], acc.at[iibufs[st].at[b]], sc_sems[st]
                ).wait()

        _fire_inputs(wid, 0)

        def _chunk(g, carry):
            s = g % NBUF
            c = wid + NW * g
            for st in range(NBUF):

                @pl.when(s == st)
                def _():
                    nxt = (st + 1) % NBUF

                    @pl.when(g >= 2)
                    def _():
                        _drain_scatter(nxt)

                    _wait_inputs(c, st)

                    @pl.when(g + 1 < nch)
                    def _():
                        _fire_inputs(c + NW, nxt)

                    dbuf, iibuf, ijbuf, ebuf = (
                        dbufs[st], iibufs[st], ijbufs[st], ebufs[st])

                    def _vec(r, carry2):
                        for t in range(128 // LANES):
                            col = t * LANES
                            off = r * 128 + col
                            d = dbuf[pl.ds(off, LANES)]
                            ii = iibuf[r, pl.ds(col, LANES)]
                            ij = ijbuf[pl.ds(off, LANES)]
                            kqi = plsc.load_gather(tab, [ii])
                            qj = plsc.load_gather(tab, [ij])
                            ebuf[r, pl.ds(col, LANES)] = _edge_energy(d, kqi, qj)
                        return carry2

                    lax.fori_loop(0, ROWS, _vec, 0)

                    for b in range(ROWS):
                        pltpu.async_copy(
                            ebuf.at[b], acc.at[iibuf.at[b]], sc_sems[st],
                            add=True)

            return carry

        lax.fori_loop(0, nch, _chunk, 0)

        for st in range(NBUF):
            @pl.when((nch - 2) % NBUF == st)
            def _():
                _drain_scatter(st)

            @pl.when((nch - 1) % NBUF == st)
            def _():
                _drain_scatter(st)

        plsc.subcore_barrier()
        pltpu.sync_copy(acc.at[pl.ds(sid * slc, slc)],
                        out_hbm.at[cid * NS + sid, 0])

    return pl.kernel(
        body,
        out_type=jax.ShapeDtypeStruct((NW, 1, slc), jnp.float32),
        mesh=mesh,
        compiler_params=pltpu.CompilerParams(needs_layout_passes=False),
        scratch_types=[
            pltpu.VMEM((n_nodes,), jnp.float32),
            [pltpu.VMEM((CHUNK,), jnp.float32) for _ in range(NBUF)],
            [pltpu.VMEM((ROWS, 128), jnp.int32) for _ in range(NBUF)],
            [pltpu.VMEM((CHUNK,), jnp.int32) for _ in range(NBUF)],
            [pltpu.VMEM((ROWS, 128), jnp.float32) for _ in range(NBUF)],
            pltpu.VMEM((zlen,), jnp.float32),
            pltpu.VMEM_SHARED((acc_pad,), jnp.float32),
            [pltpu.SemaphoreType.DMA for _ in range(NBUF)],
            [pltpu.SemaphoreType.DMA for _ in range(NBUF)],
        ],
    )


@jax.jit
def kernel(atomic_charges, distances, idx_i, idx_j):
    n_nodes = atomic_charges.shape[0]
    n_edges = distances.shape[0]
    q = atomic_charges.astype(jnp.float32)
    d = distances.astype(jnp.float32)
    ii = idx_i.astype(jnp.int32).reshape(n_edges // 128, 128)
    ij = idx_j.astype(jnp.int32)
    partials = _make_sc_kernel(n_nodes, n_edges)(q, d, ii, ij)
    per_core = partials.reshape(NC, NS * partials.shape[2])
    return (per_core[0] + per_core[1])[:n_nodes]


# speedup vs baseline: 715.2854x; 1.0023x over previous
import functools

import jax
import jax.numpy as jnp
from jax import lax
from jax.experimental import pallas as pl
from jax.experimental.pallas import tpu as pltpu
from jax.experimental.pallas import tpu_sc as plsc

NC = 2
NS = 16
NW = NC * NS
LANES = 16
CHUNK = 1024
ROWS = CHUNK // 128
NBUF = 3

KEHALF = 7.199822675975274
CUTOFF = 12.0
CUTOFF_SQ = CUTOFF * CUTOFF
CUTOFF_SR = 8.0


def _edge_energy(d, qi, qj):
    A = KEHALF
    C = KEHALF / CUTOFF_SQ
    S = 2.0 * KEHALF / CUTOFF
    a = d * d + 1.0
    bits = lax.bitcast_convert_type(a, jnp.int32)
    y = lax.bitcast_convert_type(jnp.int32(0x5F3759DF) - (bits >> 1), jnp.float32)
    h = 0.5 * a
    y = y * (1.5 - h * y * y)
    f_sh = y * (A + a * C)
    f_or = A / d + d * C
    x = jnp.minimum(d, CUTOFF_SR) * (1.0 / CUTOFF_SR)
    x2 = x * x
    poly = 6.0 * x2 - 15.0 * x + 10.0
    sw_off = 1.0 - (x2 * x) * poly
    t = (f_or - S) + sw_off * (f_sh - f_or)
    e = qi * qj * t
    return jnp.where(d <= CUTOFF, e, jnp.zeros_like(e))


def _make_sc_kernel(n_nodes, n_edges):
    assert n_edges % CHUNK == 0
    acc_pad = ((n_nodes + NS * 128 - 1) // (NS * 128)) * (NS * 128)
    slc = acc_pad // NS
    zlen = slc // 4
    assert zlen % LANES == 0
    nchunks = n_edges // CHUNK
    base_ch = nchunks // NW
    rem_ch = nchunks % NW

    mesh = plsc.VectorSubcoreMesh(
        core_axis_name="c", subcore_axis_name="s", num_cores=NC, num_subcores=NS
    )

    def body(q_hbm, d_hbm, ii_hbm, ij_hbm, out_hbm,
             tab, dbufs, iibufs, ijbufs, ebufs, zbuf, acc, in_sems, sc_sems):
        cid = lax.axis_index("c")
        sid = lax.axis_index("s")
        wid = sid * NC + cid

        pltpu.sync_copy(q_hbm, tab)

        zz = jnp.zeros((LANES,), jnp.float32)

        def _z(i, carry):
            zbuf[pl.ds(i * LANES, LANES)] = zz
            return carry

        lax.fori_loop(0, zlen // LANES, _z, 0)
        for p in range(4):
            pltpu.sync_copy(zbuf, acc.at[pl.ds(sid * slc + p * zlen, zlen)])
        plsc.subcore_barrier()

        nch = base_ch + jnp.where(wid < rem_ch, 1, 0)

        def _in_copies(c, st):
            base = c * CHUNK
            rbase = c * ROWS
            return (
                pltpu.make_async_copy(
                    d_hbm.at[pl.ds(base, CHUNK)], dbufs[st], in_sems[st]),
                pltpu.make_async_copy(
                    ii_hbm.at[pl.ds(rbase, ROWS)], iibufs[st], in_sems[st]),
                pltpu.make_async_copy(
                    ij_hbm.at[pl.ds(base, CHUNK)], ijbufs[st], in_sems[st]),
            )

        def _fire_inputs(c, st):
            for cp in _in_copies(c, st):
                cp.start()

        def _wait_inputs(c, st):
            for cp in _in_copies(c, st):
                cp.wait()

        def _drain_scatter(st):
            for b in range(ROWS):
                pltpu.make_async_copy(
                    ebufs[st].at[b], acc.at[iibufs[st].at[b]], sc_sems[st]
                ).wait()

        _fire_inputs(wid, 0)

        def _chunk(g, carry):
            s = g % NBUF
            c = wid + NW * g
            for st in range(NBUF):

                @pl.when(s == st)
                def _():
                    nxt = (st + 1) % NBUF

                    @pl.when(g >= 2)
                    def _():
                        _drain_scatter(nxt)

                    _wait_inputs(c, st)

                    @pl.when(g + 1 < nch)
                    def _():
                        _fire_inputs(c + NW, nxt)

                    dbuf, iibuf, ijbuf, ebuf = (
                        dbufs[st], iibufs[st], ijbufs[st], ebufs[st])

                    def _vec(r, carry2):
                        for t in range(128 // LANES):
                            col = t * LANES
                            off = r * 128 + col
                            d = dbuf[pl.ds(off, LANES)]
                            ii = iibuf[r, pl.ds(col, LANES)]
                            ij = ijbuf[pl.ds(off, LANES)]
                            kqi = plsc.load_gather(tab, [ii])
                            qj = plsc.load_gather(tab, [ij])
                            ebuf[r, pl.ds(col, LANES)] = _edge_energy(d, kqi, qj)
                        return carry2

                    lax.fori_loop(0, ROWS, _vec, 0)

                    for b in range(ROWS):
                        pltpu.async_copy(
                            ebuf.at[b], acc.at[iibuf.at[b]], sc_sems[st],
                            add=True)

            return carry

        lax.fori_loop(0, nch, _chunk, 0)

        for st in range(NBUF):
            @pl.when((nch - 2) % NBUF == st)
            def _():
                _drain_scatter(st)

            @pl.when((nch - 1) % NBUF == st)
            def _():
                _drain_scatter(st)

        plsc.subcore_barrier()
        pltpu.sync_copy(acc.at[pl.ds(sid * slc, slc)],
                        out_hbm.at[cid * NS + sid, 0])

    return pl.kernel(
        body,
        out_type=jax.ShapeDtypeStruct((NW, 1, slc), jnp.float32),
        mesh=mesh,
        compiler_params=pltpu.CompilerParams(needs_layout_passes=False),
        scratch_types=[
            pltpu.VMEM((n_nodes,), jnp.float32),
            [pltpu.VMEM((CHUNK,), jnp.float32) for _ in range(NBUF)],
            [pltpu.VMEM((ROWS, 128), jnp.int32) for _ in range(NBUF)],
            [pltpu.VMEM((CHUNK,), jnp.int32) for _ in range(NBUF)],
            [pltpu.VMEM((ROWS, 128), jnp.float32) for _ in range(NBUF)],
            pltpu.VMEM((zlen,), jnp.float32),
            pltpu.VMEM_SHARED((acc_pad,), jnp.float32),
            [pltpu.SemaphoreType.DMA for _ in range(NBUF)],
            [pltpu.SemaphoreType.DMA for _ in range(NBUF)],
        ],
    )


@jax.jit
def kernel(atomic_charges, distances, idx_i, idx_j):
    n_nodes = atomic_charges.shape[0]
    n_edges = distances.shape[0]
    q = atomic_charges.astype(jnp.float32)
    d = distances.astype(jnp.float32)
    ii = idx_i.astype(jnp.int32).reshape(n_edges // 128, 128)
    ij = idx_j.astype(jnp.int32)
    partials = _make_sc_kernel(n_nodes, n_edges)(q, d, ii, ij)
    per_core = partials.reshape(NC, NS * partials.shape[2])
    return (per_core[0] + per_core[1])[:n_nodes]
